# baseline (speedup 1.0000x reference)
import jax
import jax.numpy as jnp
from jax.experimental import pallas as pl

_BLK_ROWS = 1024
_PK = 512


def _mask_kernel(mtop_ref, out_ref):
    mtop = mtop_ref[...]
    j8 = jax.lax.broadcasted_iota(jnp.int32, out_ref.shape, 1) * 8
    t = jnp.clip(mtop + 1 - j8, 0, 8)
    out_ref[...] = ((0xFF << t) & 0xFF).astype(jnp.uint8)


def kernel(m_top, scores):
    B, H, U, L_K = scores.shape
    rows = B * H * U
    grid = rows // _BLK_ROWS
    mt = m_top.reshape(rows, 1).astype(jnp.int32)
    packed = pl.pallas_call(
        _mask_kernel,
        grid=(grid,),
        in_specs=[pl.BlockSpec((_BLK_ROWS, 1), lambda i: (i, 0))],
        out_specs=pl.BlockSpec((_BLK_ROWS, _PK), lambda i: (i, 0)),
        out_shape=jax.ShapeDtypeStruct((rows, _PK), jnp.uint8),
    )(mt)
    bits = jnp.arange(8, dtype=jnp.uint8)
    out = ((packed[..., None] >> bits) & 1).astype(jnp.bool_)
    return out.reshape(B, H, U, L_K)


# speedup vs baseline: 1.7009x; 1.7009x over previous
import jax
import jax.numpy as jnp
from jax.experimental import pallas as pl

_BLK_ROWS = 1024
_PK = 512


def _mask_kernel(mtop_ref, out_ref):
    mtop = mtop_ref[...]
    j = jax.lax.broadcasted_iota(jnp.int32, out_ref.shape, 1)
    t = jnp.clip(jax.lax.shift_right_arithmetic(mtop - j + 512, 9), 0, 8)
    out_ref[...] = ((0xFF << t) & 0xFF).astype(jnp.uint8)


def kernel(m_top, scores):
    B, H, U, L_K = scores.shape
    rows = B * H * U
    grid = rows // _BLK_ROWS
    mt = m_top.reshape(rows, 1).astype(jnp.int32)
    packed = pl.pallas_call(
        _mask_kernel,
        grid=(grid,),
        in_specs=[pl.BlockSpec((_BLK_ROWS, 1), lambda i: (i, 0))],
        out_specs=pl.BlockSpec((_BLK_ROWS, _PK), lambda i: (i, 0)),
        out_shape=jax.ShapeDtypeStruct((rows, _PK), jnp.uint8),
    )(mt)
    bits = jnp.arange(8, dtype=jnp.uint8).reshape(1, 8, 1)
    out = ((packed[:, None, :] >> bits) & 1).astype(jnp.bool_)
    return out.reshape(B, H, U, L_K)


# speedup vs baseline: 6.2683x; 3.6852x over previous
import jax
import jax.numpy as jnp
from jax.experimental import pallas as pl

_BLK_ROWS = 1024


def _mask_kernel(mtop_ref, out_ref):
    mtop = mtop_ref[...]
    cols = jax.lax.broadcasted_iota(jnp.int32, out_ref.shape, 1)
    out_ref[...] = (cols > mtop).astype(jnp.int8)


def kernel(m_top, scores):
    B, H, U, L_K = scores.shape
    rows = B * H * U
    grid = rows // _BLK_ROWS
    mt = m_top.reshape(rows, 1).astype(jnp.int32)
    out = pl.pallas_call(
        _mask_kernel,
        grid=(grid,),
        in_specs=[pl.BlockSpec((_BLK_ROWS, 1), lambda i: (i, 0))],
        out_specs=pl.BlockSpec((_BLK_ROWS, L_K), lambda i: (i, 0)),
        out_shape=jax.ShapeDtypeStruct((rows, L_K), jnp.int8),
    )(mt)
    return out.reshape(B, H, U, L_K).astype(jnp.bool_)
